# baseline (speedup 1.0000x reference)
import jax
import jax.numpy as jnp
from jax.experimental import pallas as pl
from jax.experimental.pallas import tpu as pltpu

_N_OUT = 100
_NC_PER_TASK = 10
_NEG_FILL = -100000000000.0
_BM = 512


def _fused_linear_mask_kernel(t_ref, x_ref, w_ref, b_ref, o_ref):
    off1 = t_ref[0] * _NC_PER_TASK
    off2 = off1 + _NC_PER_TASK
    xb = x_ref[...].astype(jnp.bfloat16)
    wb = w_ref[...].astype(jnp.bfloat16)
    acc = jnp.dot(xb, wb, preferred_element_type=jnp.float32)
    cols = jax.lax.broadcasted_iota(jnp.int32, (1, _N_OUT), 1)
    keep = (cols >= off1) & (cols < off2)
    o_ref[...] = jnp.where(keep, acc + b_ref[...], _NEG_FILL)


def kernel(x, W, b, t):
    B = x.shape[0]
    x2 = x.reshape(B, -1)
    K = x2.shape[1]
    t_arr = jnp.atleast_1d(jnp.asarray(t, jnp.int32))
    b2 = b.reshape(1, _N_OUT)
    grid = (B // _BM,)
    return pl.pallas_call(
        _fused_linear_mask_kernel,
        grid_spec=pltpu.PrefetchScalarGridSpec(
            num_scalar_prefetch=1,
            grid=grid,
            in_specs=[
                pl.BlockSpec((_BM, K), lambda i, t_s: (i, 0)),
                pl.BlockSpec((K, _N_OUT), lambda i, t_s: (0, 0)),
                pl.BlockSpec((1, _N_OUT), lambda i, t_s: (0, 0)),
            ],
            out_specs=pl.BlockSpec((_BM, _N_OUT), lambda i, t_s: (i, 0)),
        ),
        out_shape=jax.ShapeDtypeStruct((B, _N_OUT), jnp.float32),
        compiler_params=pltpu.CompilerParams(
            dimension_semantics=("arbitrary",),
        ),
    )(t_arr, x2, W, b2)
